# baseline (speedup 1.0000x reference)
import functools

import jax
import jax.numpy as jnp
from jax import lax
from jax.experimental import pallas as pl
from jax.experimental.pallas import tpu as pltpu
from jax.experimental.pallas import tpu_sc as plsc

K = 16
QB = 128
NC, NS, L = 2, 16, 16
NW = NC * NS
CAND = 128
GROUP = 8


def _sum16(x):
    v = x
    while v.shape[1] > 1:
        h = v.shape[1] // 2
        v = v[:, :h] + v[:, h:]
    return v[:, 0]


def _dist_block(query_ref, ref_ref, d2_ref, t_ref):
    q = query_ref[0]
    r = ref_ref[0]
    n = r.shape[0]
    dot = lax.dot_general(q, r, (((1,), (1,)), ((), ())),
                          preferred_element_type=jnp.float32)
    q2 = _sum16(q * q)
    r2 = _sum16(r * r)
    d2 = jnp.maximum((q2[:, None] + r2[None, :]) - 2.0 * dot, 0.0)
    d2_ref[0] = d2
    lm = d2[:, 0:128]
    for i in range(1, n // 128):
        lm = jnp.minimum(lm, d2[:, i * 128:(i + 1) * 128])
    inf = jnp.float32(jnp.inf)
    for _ in range(K - 1):
        m = jnp.min(lm, axis=1)
        lm = jnp.where(lm == m[:, None], inf, lm)
    t_ref[0] = jnp.broadcast_to(jnp.min(lm, axis=1)[:, None], (lm.shape[0], K))


def _stage_a(ref, query):
    b, n, d = ref.shape
    _, q, _ = query.shape
    return pl.pallas_call(
        _dist_block,
        grid=(b, q // QB),
        in_specs=[
            pl.BlockSpec((1, QB, d), lambda bi, qi: (bi, qi, 0)),
            pl.BlockSpec((1, n, d), lambda bi, qi: (bi, 0, 0)),
        ],
        out_specs=[
            pl.BlockSpec((1, QB, n), lambda bi, qi: (bi, qi, 0)),
            pl.BlockSpec((1, QB, K), lambda bi, qi: (bi, qi, 0)),
        ],
        out_shape=[
            jax.ShapeDtypeStruct((b, q, n), jnp.float32),
            jax.ShapeDtypeStruct((b, q, K), jnp.float32),
        ],
    )(query, ref)


def _gather16(x, idx):
    return lax.gather(
        x, idx[:, None],
        dimension_numbers=lax.GatherDimensionNumbers(
            offset_dims=(), collapsed_slice_dims=(0,), start_index_map=(0,)),
        slice_sizes=(1,),
        mode=lax.GatherScatterMode.PROMISE_IN_BOUNDS)


def _sc_body(d2_hbm, t_hbm, vals_hbm, idx_hbm,
             tv, buf0, buf1, cv, ci, ov, oi, sem0, sem1):
    n = d2_hbm.shape[1]
    rows_per = d2_hbm.shape[0] // NW
    wid = lax.axis_index("s") * NC + lax.axis_index("c")
    base = wid * rows_per

    pltpu.sync_copy(t_hbm.at[pl.ds(base * K, rows_per * K)], tv)

    iota = lax.broadcasted_iota(jnp.int32, (L,), 0)
    inf = jnp.float32(jnp.inf)
    inf_vec = jnp.full((L,), inf, jnp.float32)
    big_i = jnp.full((L,), jnp.int32(2 ** 30), jnp.int32)
    n_groups = n // (GROUP * L)

    def start(row, buf, sem):
        pltpu.make_async_copy(d2_hbm.at[base + row], buf, sem).start()

    def wait(row, buf, sem):
        pltpu.make_async_copy(d2_hbm.at[base + row], buf, sem).wait()

    def process_row(rl, buf):
        t_vec = tv[pl.ds(rl * K, L)]
        for j in range(CAND // L):
            cv[pl.ds(j * L, L)] = inf_vec

        def scan_group(g, cnt):
            e0 = g * (GROUP * L)
            gm = buf[pl.ds(e0, L)]
            for u in range(1, GROUP):
                gm = jnp.minimum(gm, buf[pl.ds(e0 + u * L, L)])

            def do_hits(c):
                vs, ms, pcs = [], [], []
                for u in range(GROUP):
                    v = buf[pl.ds(e0 + u * L, L)]
                    m = v <= t_vec
                    vs.append(v)
                    ms.append(m)
                    pcs.append(jnp.sum(m.astype(jnp.int32), axis=0))
                for u in range(GROUP):
                    idxv = iota + (e0 + u * L)
                    off = jnp.minimum(c, CAND - L)
                    plsc.store_compressed(cv.at[pl.ds(off, L)], vs[u],
                                          mask=ms[u])
                    plsc.store_compressed(ci.at[pl.ds(off, L)], idxv,
                                          mask=ms[u])
                    c = c + pcs[u]
                return c

            return lax.cond(jnp.any(gm <= t_vec), do_hits, lambda c: c, cnt)

        cnt = lax.fori_loop(0, n_groups, scan_group, jnp.int32(0))

        def merge(j, carry):
            acc_v, acc_i = carry
            v = cv[pl.ds(j * L, L)]
            i = ci[pl.ds(j * L, L)]
            sv, si = plsc.sort_key_val(v, i)
            rv = lax.rev(sv, (0,))
            ri = lax.rev(si, (0,))
            keep = (acc_v < rv) | ((acc_v == rv) & (acc_i <= ri))
            mv = jnp.where(keep, acc_v, rv)
            mi = jnp.where(keep, acc_i, ri)
            return tuple(plsc.sort_key_val(mv, mi))

        nv = (cnt + (L - 1)) // L
        acc_v, acc_i = lax.fori_loop(0, nv, merge, (inf_vec, big_i))
        nxt = jnp.minimum(iota + 1, L - 1)
        prv = jnp.maximum(iota - 1, 0)
        vn = _gather16(acc_v, nxt)
        ix = _gather16(acc_i, nxt)
        vp = _gather16(acc_v, prv)
        ip = _gather16(acc_i, prv)
        take_next = (vn == acc_v) & (ix < acc_i)
        take_prev = (vp == acc_v) & (ip > acc_i)
        acc_i = jnp.where(take_next, ix, jnp.where(take_prev, ip, acc_i))
        ov[pl.ds(rl * K, K)] = acc_v
        oi[pl.ds(rl * K, K)] = acc_i

    start(0, buf0, sem0)
    start(1, buf1, sem1)

    def pair(g, _):
        row = 2 * g
        wait(row, buf0, sem0)
        process_row(row, buf0)

        @pl.when(g < rows_per // 2 - 1)
        def _():
            start(row + 2, buf0, sem0)

        wait(row + 1, buf1, sem1)
        process_row(row + 1, buf1)

        @pl.when(g < rows_per // 2 - 1)
        def _():
            start(row + 3, buf1, sem1)

        return 0

    lax.fori_loop(0, rows_per // 2, pair, 0)

    pltpu.sync_copy(ov, vals_hbm.at[pl.ds(base * K, rows_per * K)])
    pltpu.sync_copy(oi, idx_hbm.at[pl.ds(base * K, rows_per * K)])


def _stage_b(d2, t):
    rows, n = d2.shape
    rows_per = rows // NW
    mesh = plsc.VectorSubcoreMesh(core_axis_name="c", subcore_axis_name="s",
                                  num_cores=NC, num_subcores=NS)
    f = pl.kernel(
        _sc_body,
        out_type=[
            jax.ShapeDtypeStruct((rows * K,), jnp.float32),
            jax.ShapeDtypeStruct((rows * K,), jnp.int32),
        ],
        mesh=mesh,
        scratch_types=[
            pltpu.VMEM((rows_per * K,), jnp.float32),
            pltpu.VMEM((n,), jnp.float32),
            pltpu.VMEM((n,), jnp.float32),
            pltpu.VMEM((CAND,), jnp.float32),
            pltpu.VMEM((CAND,), jnp.int32),
            pltpu.VMEM((rows_per * K,), jnp.float32),
            pltpu.VMEM((rows_per * K,), jnp.int32),
            pltpu.SemaphoreType.DMA,
            pltpu.SemaphoreType.DMA,
        ],
        compiler_params=pltpu.CompilerParams(needs_layout_passes=False),
    )
    return f(d2, t)


def _sqrt_body(v_ref, o_ref):
    o_ref[...] = jnp.sqrt(v_ref[...])


def _stage_c(v):
    return pl.pallas_call(
        _sqrt_body,
        out_shape=jax.ShapeDtypeStruct(v.shape, jnp.float32),
    )(v)


def kernel(ref, query):
    b, n, d = ref.shape
    _, q, _ = query.shape
    d2, t = _stage_a(ref, query)
    vals, idxs = _stage_b(d2.reshape(b * q, n), t.reshape(b * q * K))
    dist = _stage_c(vals.reshape(b * q, K))
    return dist.reshape(b, q, K), idxs.reshape(b, q, K)


# speedup vs baseline: 1.7053x; 1.7053x over previous
import functools

import jax
import jax.numpy as jnp
from jax import lax
from jax.experimental import pallas as pl
from jax.experimental.pallas import tpu as pltpu
from jax.experimental.pallas import tpu_sc as plsc

K = 16
QB = 128
NC, NS, L = 2, 16, 16
NW = NC * NS
CAND = 128
GROUP = 16


def _sum16(x):
    v = x
    while v.shape[1] > 1:
        h = v.shape[1] // 2
        v = v[:, :h] + v[:, h:]
    return v[:, 0]


def _r2_block(ref_ref, r2_ref):
    r = ref_ref[0]
    r2_ref[0, 0] = _sum16(r * r)


def _stage_a0(ref):
    b, n, d = ref.shape
    return pl.pallas_call(
        _r2_block,
        grid=(b,),
        in_specs=[pl.BlockSpec((1, n, d), lambda bi: (bi, 0, 0))],
        out_specs=pl.BlockSpec((1, 1, n), lambda bi: (bi, 0, 0)),
        out_shape=jax.ShapeDtypeStruct((b, 1, n), jnp.float32),
    )(ref)


def _dist_block(query_ref, ref_ref, r2_ref, d2_ref, t_ref):
    q = query_ref[0]
    r = ref_ref[0]
    n = r.shape[0]
    dot = lax.dot_general(q, r, (((1,), (1,)), ((), ())),
                          preferred_element_type=jnp.float32)
    q2 = _sum16(q * q)
    r2 = r2_ref[0, 0]
    d2 = jnp.maximum((q2[:, None] + r2[None, :]) - 2.0 * dot, 0.0)
    d2_ref[0] = d2
    lm = d2[:, 0:128]
    for i in range(1, n // 128):
        lm = jnp.minimum(lm, d2[:, i * 128:(i + 1) * 128])
    inf = jnp.float32(jnp.inf)
    for _ in range(K - 1):
        m = jnp.min(lm, axis=1)
        lm = jnp.where(lm == m[:, None], inf, lm)
    t_ref[0] = jnp.broadcast_to(jnp.min(lm, axis=1)[:, None], (lm.shape[0], K))


QBA = 128


def _stage_a(ref, query, r2):
    b, n, d = ref.shape
    _, q, _ = query.shape
    return pl.pallas_call(
        _dist_block,
        grid=(b, q // QBA),
        in_specs=[
            pl.BlockSpec((1, QBA, d), lambda bi, qi: (bi, qi, 0)),
            pl.BlockSpec((1, n, d), lambda bi, qi: (bi, 0, 0)),
            pl.BlockSpec((1, 1, n), lambda bi, qi: (bi, 0, 0)),
        ],
        out_specs=[
            pl.BlockSpec((1, QBA, n), lambda bi, qi: (bi, qi, 0)),
            pl.BlockSpec((1, QBA, K), lambda bi, qi: (bi, qi, 0)),
        ],
        out_shape=[
            jax.ShapeDtypeStruct((b, q, n), jnp.float32),
            jax.ShapeDtypeStruct((b, q, K), jnp.float32),
        ],
    )(query, ref, r2)


def _gather16(x, idx):
    return lax.gather(
        x, idx[:, None],
        dimension_numbers=lax.GatherDimensionNumbers(
            offset_dims=(), collapsed_slice_dims=(0,), start_index_map=(0,)),
        slice_sizes=(1,),
        mode=lax.GatherScatterMode.PROMISE_IN_BOUNDS)


def _sc_body(d2_hbm, t_hbm, vals_hbm, idx_hbm,
             tv, buf0, buf1, cv, ci, ov, oi, sem0, sem1):
    n = d2_hbm.shape[1]
    rows_per = d2_hbm.shape[0] // NW
    wid = lax.axis_index("s") * NC + lax.axis_index("c")
    base = wid * rows_per

    pltpu.sync_copy(t_hbm.at[pl.ds(base * K, rows_per * K)], tv)

    iota = lax.broadcasted_iota(jnp.int32, (L,), 0)
    inf = jnp.float32(jnp.inf)
    inf_vec = jnp.full((L,), inf, jnp.float32)
    big_i = jnp.full((L,), jnp.int32(2 ** 30), jnp.int32)
    n_groups = n // (GROUP * L)

    def start(row, buf, sem):
        pltpu.make_async_copy(d2_hbm.at[base + row], buf, sem).start()

    def wait(row, buf, sem):
        pltpu.make_async_copy(d2_hbm.at[base + row], buf, sem).wait()

    def process_row(rl, buf):
        t_vec = tv[pl.ds(rl * K, L)]
        for j in range(CAND // L):
            cv[pl.ds(j * L, L)] = inf_vec

        def scan_group(g, cnt):
            e0 = g * (GROUP * L)
            vs = [buf[pl.ds(e0 + u * L, L)] for u in range(GROUP)]
            tier = vs
            while len(tier) > 1:
                tier = [jnp.minimum(a, b) for a, b in zip(tier[::2], tier[1::2])]
            gm = tier[0]

            def do_hits(c):
                vs, ms, pcs = [], [], []
                for u in range(GROUP):
                    v = buf[pl.ds(e0 + u * L, L)]
                    m = v <= t_vec
                    vs.append(v)
                    ms.append(m)
                    pcs.append(jnp.sum(m.astype(jnp.int32), axis=0))
                for u in range(GROUP):
                    idxv = iota + (e0 + u * L)
                    off = jnp.minimum(c, CAND - L)
                    plsc.store_compressed(cv.at[pl.ds(off, L)], vs[u],
                                          mask=ms[u])
                    plsc.store_compressed(ci.at[pl.ds(off, L)], idxv,
                                          mask=ms[u])
                    c = c + pcs[u]
                return c

            return lax.cond(jnp.any(gm <= t_vec), do_hits, lambda c: c, cnt)

        cnt = lax.fori_loop(0, n_groups, scan_group, jnp.int32(0))

        def merge(j, carry):
            acc_v, acc_i = carry
            v = cv[pl.ds(j * L, L)]
            i = ci[pl.ds(j * L, L)]
            sv, si = plsc.sort_key_val(v, i)
            rv = lax.rev(sv, (0,))
            ri = lax.rev(si, (0,))
            keep = (acc_v < rv) | ((acc_v == rv) & (acc_i <= ri))
            mv = jnp.where(keep, acc_v, rv)
            mi = jnp.where(keep, acc_i, ri)
            return tuple(plsc.sort_key_val(mv, mi))

        nv = (cnt + (L - 1)) // L
        acc_v, acc_i = lax.fori_loop(0, nv, merge, (inf_vec, big_i))
        nxt = jnp.minimum(iota + 1, L - 1)
        prv = jnp.maximum(iota - 1, 0)
        vn = _gather16(acc_v, nxt)
        ix = _gather16(acc_i, nxt)
        vp = _gather16(acc_v, prv)
        ip = _gather16(acc_i, prv)
        take_next = (vn == acc_v) & (ix < acc_i)
        take_prev = (vp == acc_v) & (ip > acc_i)
        acc_i = jnp.where(take_next, ix, jnp.where(take_prev, ip, acc_i))
        ov[pl.ds(rl * K, K)] = acc_v
        oi[pl.ds(rl * K, K)] = acc_i

    start(0, buf0, sem0)
    start(1, buf1, sem1)

    def pair(g, _):
        row = 2 * g
        wait(row, buf0, sem0)
        process_row(row, buf0)

        @pl.when(g < rows_per // 2 - 1)
        def _():
            start(row + 2, buf0, sem0)

        wait(row + 1, buf1, sem1)
        process_row(row + 1, buf1)

        @pl.when(g < rows_per // 2 - 1)
        def _():
            start(row + 3, buf1, sem1)

        return 0

    lax.fori_loop(0, rows_per // 2, pair, 0)

    pltpu.sync_copy(ov, vals_hbm.at[pl.ds(base * K, rows_per * K)])
    pltpu.sync_copy(oi, idx_hbm.at[pl.ds(base * K, rows_per * K)])


def _stage_b(d2, t):
    rows, n = d2.shape
    rows_per = rows // NW
    mesh = plsc.VectorSubcoreMesh(core_axis_name="c", subcore_axis_name="s",
                                  num_cores=NC, num_subcores=NS)
    f = pl.kernel(
        _sc_body,
        out_type=[
            jax.ShapeDtypeStruct((rows * K,), jnp.float32),
            jax.ShapeDtypeStruct((rows * K,), jnp.int32),
        ],
        mesh=mesh,
        scratch_types=[
            pltpu.VMEM((rows_per * K,), jnp.float32),
            pltpu.VMEM((n,), jnp.float32),
            pltpu.VMEM((n,), jnp.float32),
            pltpu.VMEM((CAND,), jnp.float32),
            pltpu.VMEM((CAND,), jnp.int32),
            pltpu.VMEM((rows_per * K,), jnp.float32),
            pltpu.VMEM((rows_per * K,), jnp.int32),
            pltpu.SemaphoreType.DMA,
            pltpu.SemaphoreType.DMA,
        ],
        compiler_params=pltpu.CompilerParams(needs_layout_passes=False),
    )
    return f(d2, t)


def _sqrt_body(v_ref, o_ref):
    o_ref[...] = jnp.sqrt(v_ref[...])


def _stage_c(v):
    return pl.pallas_call(
        _sqrt_body,
        out_shape=jax.ShapeDtypeStruct(v.shape, jnp.float32),
    )(v)


def kernel(ref, query):
    b, n, d = ref.shape
    _, q, _ = query.shape
    d2, t = _stage_a(ref, query, _stage_a0(ref))
    vals, idxs = _stage_b(d2.reshape(b * q, n), t.reshape(b * q * K))
    dist = _stage_c(vals.reshape(b * q, K))
    return dist.reshape(b, q, K), idxs.reshape(b, q, K)


# speedup vs baseline: 1.7912x; 1.0504x over previous
import functools

import jax
import jax.numpy as jnp
from jax import lax
from jax.experimental import pallas as pl
from jax.experimental.pallas import tpu as pltpu
from jax.experimental.pallas import tpu_sc as plsc

K = 16
QB = 128
NC, NS, L = 2, 16, 16
NW = NC * NS
CAND = 128
GROUP = 16


def _sum16(x):
    v = x
    while v.shape[1] > 1:
        h = v.shape[1] // 2
        v = v[:, :h] + v[:, h:]
    return v[:, 0]


def _r2_block(ref_ref, r2_ref):
    r = ref_ref[0]
    r2_ref[0, 0] = _sum16(r * r)


def _stage_a0(ref):
    b, n, d = ref.shape
    return pl.pallas_call(
        _r2_block,
        grid=(b,),
        in_specs=[pl.BlockSpec((1, n, d), lambda bi: (bi, 0, 0))],
        out_specs=pl.BlockSpec((1, 1, n), lambda bi: (bi, 0, 0)),
        out_shape=jax.ShapeDtypeStruct((b, 1, n), jnp.float32),
    )(ref)


def _dist_block(query_ref, ref_ref, r2_ref, d2_ref, t_ref):
    q = query_ref[0]
    r = ref_ref[0]
    n = r.shape[0]
    dot = lax.dot_general(q, r, (((1,), (1,)), ((), ())),
                          preferred_element_type=jnp.float32)
    q2 = _sum16(q * q)
    r2 = r2_ref[0, 0]
    d2 = jnp.maximum((q2[:, None] + r2[None, :]) - 2.0 * dot, 0.0)
    d2_ref[0] = d2
    lm = d2[:, 0:128]
    for i in range(1, n // 128):
        lm = jnp.minimum(lm, d2[:, i * 128:(i + 1) * 128])
    inf = jnp.float32(jnp.inf)
    for _ in range(K - 1):
        m = jnp.min(lm, axis=1)
        lm = jnp.where(lm == m[:, None], inf, lm)
    t_ref[0] = jnp.broadcast_to(jnp.min(lm, axis=1)[:, None], (lm.shape[0], K))


QBA = 128


def _stage_a(ref, query, r2):
    b, n, d = ref.shape
    _, q, _ = query.shape
    return pl.pallas_call(
        _dist_block,
        grid=(b, q // QBA),
        in_specs=[
            pl.BlockSpec((1, QBA, d), lambda bi, qi: (bi, qi, 0)),
            pl.BlockSpec((1, n, d), lambda bi, qi: (bi, 0, 0)),
            pl.BlockSpec((1, 1, n), lambda bi, qi: (bi, 0, 0)),
        ],
        out_specs=[
            pl.BlockSpec((1, QBA, n), lambda bi, qi: (bi, qi, 0)),
            pl.BlockSpec((1, QBA, K), lambda bi, qi: (bi, qi, 0)),
        ],
        out_shape=[
            jax.ShapeDtypeStruct((b, q, n), jnp.float32),
            jax.ShapeDtypeStruct((b, q, K), jnp.float32),
        ],
    )(query, ref, r2)


def _gather16(x, idx):
    return lax.gather(
        x, idx[:, None],
        dimension_numbers=lax.GatherDimensionNumbers(
            offset_dims=(), collapsed_slice_dims=(0,), start_index_map=(0,)),
        slice_sizes=(1,),
        mode=lax.GatherScatterMode.PROMISE_IN_BOUNDS)


def _sc_body(d2_hbm, t_hbm, vals_hbm, idx_hbm,
             tv, buf0, buf1, cv, ci, ov, oi, sem0, sem1):
    n = d2_hbm.shape[1]
    rows_per = d2_hbm.shape[0] // NW
    wid = lax.axis_index("s") * NC + lax.axis_index("c")
    base = wid * rows_per

    pltpu.sync_copy(t_hbm.at[pl.ds(base * K, rows_per * K)], tv)

    iota = lax.broadcasted_iota(jnp.int32, (L,), 0)
    inf = jnp.float32(jnp.inf)
    inf_vec = jnp.full((L,), inf, jnp.float32)
    big_i = jnp.full((L,), jnp.int32(2 ** 30), jnp.int32)
    n_groups = n // (GROUP * L)

    def start(row, buf, sem):
        pltpu.make_async_copy(d2_hbm.at[base + row], buf, sem).start()

    def wait(row, buf, sem):
        pltpu.make_async_copy(d2_hbm.at[base + row], buf, sem).wait()

    def process_row(rl, buf):
        t_vec = tv[pl.ds(rl * K, L)]
        for j in range(CAND // L):
            cv[pl.ds(j * L, L)] = inf_vec

        def scan_group(g, cnt):
            e0 = g * (GROUP * L)
            vs = [buf[pl.ds(e0 + u * L, L)] for u in range(GROUP)]
            tier = vs
            while len(tier) > 1:
                tier = [jnp.minimum(a, b) for a, b in zip(tier[::2], tier[1::2])]
            gm = tier[0]

            def do_hits(c):
                vs, ms, pcs = [], [], []
                for u in range(GROUP):
                    v = buf[pl.ds(e0 + u * L, L)]
                    m = v <= t_vec
                    vs.append(v)
                    ms.append(m)
                    pcs.append(jnp.sum(m.astype(jnp.int32), axis=0))
                for u in range(GROUP):
                    idxv = iota + (e0 + u * L)
                    off = jnp.minimum(c, CAND - L)
                    plsc.store_compressed(cv.at[pl.ds(off, L)], vs[u],
                                          mask=ms[u])
                    plsc.store_compressed(ci.at[pl.ds(off, L)], idxv,
                                          mask=ms[u])
                    c = c + pcs[u]
                return c

            return lax.cond(jnp.any(gm <= t_vec), do_hits, lambda c: c, cnt)

        cnt = lax.fori_loop(0, n_groups, scan_group, jnp.int32(0))

        def merge(j, carry):
            acc_v, acc_i = carry
            v = cv[pl.ds(j * L, L)]
            i = ci[pl.ds(j * L, L)]
            sv, si = plsc.sort_key_val(v, i)
            rv = lax.rev(sv, (0,))
            ri = lax.rev(si, (0,))
            keep = (acc_v < rv) | ((acc_v == rv) & (acc_i <= ri))
            mv = jnp.where(keep, acc_v, rv)
            mi = jnp.where(keep, acc_i, ri)
            return tuple(plsc.sort_key_val(mv, mi))

        nv = (cnt + (L - 1)) // L
        acc_v, acc_i = lax.fori_loop(0, nv, merge, (inf_vec, big_i))
        nxt = jnp.minimum(iota + 1, L - 1)
        prv = jnp.maximum(iota - 1, 0)
        vn = _gather16(acc_v, nxt)
        ix = _gather16(acc_i, nxt)
        vp = _gather16(acc_v, prv)
        ip = _gather16(acc_i, prv)
        take_next = (vn == acc_v) & (ix < acc_i)
        take_prev = (vp == acc_v) & (ip > acc_i)
        acc_i = jnp.where(take_next, ix, jnp.where(take_prev, ip, acc_i))
        ov[pl.ds(rl * K, K)] = acc_v
        oi[pl.ds(rl * K, K)] = acc_i

    start(0, buf0, sem0)
    start(1, buf1, sem1)

    def pair(g, _):
        row = 2 * g
        wait(row, buf0, sem0)
        process_row(row, buf0)

        @pl.when(g < rows_per // 2 - 1)
        def _():
            start(row + 2, buf0, sem0)

        wait(row + 1, buf1, sem1)
        process_row(row + 1, buf1)

        @pl.when(g < rows_per // 2 - 1)
        def _():
            start(row + 3, buf1, sem1)

        return 0

    lax.fori_loop(0, rows_per // 2, pair, 0)

    pltpu.sync_copy(ov, vals_hbm.at[pl.ds(base * K, rows_per * K)])
    pltpu.sync_copy(oi, idx_hbm.at[pl.ds(base * K, rows_per * K)])


def _stage_b(d2, t):
    rows, n = d2.shape
    rows_per = rows // NW
    mesh = plsc.VectorSubcoreMesh(core_axis_name="c", subcore_axis_name="s",
                                  num_cores=NC, num_subcores=NS)
    f = pl.kernel(
        _sc_body,
        out_type=[
            jax.ShapeDtypeStruct((rows * K,), jnp.float32),
            jax.ShapeDtypeStruct((rows * K,), jnp.int32),
        ],
        mesh=mesh,
        scratch_types=[
            pltpu.VMEM((rows_per * K,), jnp.float32),
            pltpu.VMEM((n,), jnp.float32),
            pltpu.VMEM((n,), jnp.float32),
            pltpu.VMEM((CAND,), jnp.float32),
            pltpu.VMEM((CAND,), jnp.int32),
            pltpu.VMEM((rows_per * K,), jnp.float32),
            pltpu.VMEM((rows_per * K,), jnp.int32),
            pltpu.SemaphoreType.DMA,
            pltpu.SemaphoreType.DMA,
        ],
        compiler_params=pltpu.CompilerParams(needs_layout_passes=False),
    )
    return f(d2, t)


def _sqrt_body(v_ref, o_ref):
    o_ref[...] = jnp.sqrt(v_ref[...])


def _stage_c(v):
    return pl.pallas_call(
        _sqrt_body,
        out_shape=jax.ShapeDtypeStruct(v.shape, jnp.float32),
    )(v)


def kernel(ref, query):
    b, n, d = ref.shape
    _, q, _ = query.shape
    h = b // 2
    r2 = _stage_a0(ref)
    outs = []
    for s in (slice(0, h), slice(h, b)):
        d2, t = _stage_a(ref[s], query[s], r2[s])
        outs.append(_stage_b(d2.reshape(h * q, n), t.reshape(h * q * K)))
    vals = jnp.concatenate([o[0] for o in outs])
    idxs = jnp.concatenate([o[1] for o in outs])
    dist = _stage_c(vals.reshape(b * q, K))
    return dist.reshape(b, q, K), idxs.reshape(b, q, K)


# speedup vs baseline: 1.8281x; 1.0206x over previous
import functools

import jax
import jax.numpy as jnp
from jax import lax
from jax.experimental import pallas as pl
from jax.experimental.pallas import tpu as pltpu
from jax.experimental.pallas import tpu_sc as plsc

K = 16
QB = 128
NC, NS, L = 2, 16, 16
NW = NC * NS
CAND = 128
GROUP = 16


def _sum16(x):
    v = x
    while v.shape[1] > 1:
        h = v.shape[1] // 2
        v = v[:, :h] + v[:, h:]
    return v[:, 0]


def _r2_block(ref_ref, r2_ref):
    r = ref_ref[0]
    r2_ref[0, 0] = _sum16(r * r)


def _stage_a0(ref):
    b, n, d = ref.shape
    return pl.pallas_call(
        _r2_block,
        grid=(b,),
        in_specs=[pl.BlockSpec((1, n, d), lambda bi: (bi, 0, 0))],
        out_specs=pl.BlockSpec((1, 1, n), lambda bi: (bi, 0, 0)),
        out_shape=jax.ShapeDtypeStruct((b, 1, n), jnp.float32),
    )(ref)


def _dist_block(query_ref, ref_ref, r2_ref, d2_ref, t_ref):
    q = query_ref[0]
    r = ref_ref[0]
    n = r.shape[0]
    dot = lax.dot_general(q, r, (((1,), (1,)), ((), ())),
                          preferred_element_type=jnp.float32)
    q2 = _sum16(q * q)
    r2 = r2_ref[0, 0]
    d2 = jnp.maximum((q2[:, None] + r2[None, :]) - 2.0 * dot, 0.0)
    d2_ref[0] = d2
    lm = d2[:, 0:128]
    for i in range(1, n // 128):
        lm = jnp.minimum(lm, d2[:, i * 128:(i + 1) * 128])
    inf = jnp.float32(jnp.inf)
    for _ in range(K - 1):
        m = jnp.min(lm, axis=1)
        lm = jnp.where(lm == m[:, None], inf, lm)
    t_ref[0] = jnp.broadcast_to(jnp.min(lm, axis=1)[:, None], (lm.shape[0], K))


QBA = 128


def _stage_a(ref, query, r2):
    b, n, d = ref.shape
    _, q, _ = query.shape
    return pl.pallas_call(
        _dist_block,
        grid=(b, q // QBA),
        in_specs=[
            pl.BlockSpec((1, QBA, d), lambda bi, qi: (bi, qi, 0)),
            pl.BlockSpec((1, n, d), lambda bi, qi: (bi, 0, 0)),
            pl.BlockSpec((1, 1, n), lambda bi, qi: (bi, 0, 0)),
        ],
        out_specs=[
            pl.BlockSpec((1, QBA, n), lambda bi, qi: (bi, qi, 0)),
            pl.BlockSpec((1, QBA, K), lambda bi, qi: (bi, qi, 0)),
        ],
        out_shape=[
            jax.ShapeDtypeStruct((b, q, n), jnp.float32),
            jax.ShapeDtypeStruct((b, q, K), jnp.float32),
        ],
    )(query, ref, r2)


def _gather16(x, idx):
    return lax.gather(
        x, idx[:, None],
        dimension_numbers=lax.GatherDimensionNumbers(
            offset_dims=(), collapsed_slice_dims=(0,), start_index_map=(0,)),
        slice_sizes=(1,),
        mode=lax.GatherScatterMode.PROMISE_IN_BOUNDS)


def _sc_body(d2_hbm, t_hbm, vals_hbm, idx_hbm,
             tv, buf0, buf1, cv, ci, ov, oi, sem0, sem1):
    n = d2_hbm.shape[1]
    rows_per = d2_hbm.shape[0] // NW
    wid = lax.axis_index("s") * NC + lax.axis_index("c")
    base = wid * rows_per

    pltpu.sync_copy(t_hbm.at[pl.ds(base * K, rows_per * K)], tv)

    iota = lax.broadcasted_iota(jnp.int32, (L,), 0)
    inf = jnp.float32(jnp.inf)
    inf_vec = jnp.full((L,), inf, jnp.float32)
    big_i = jnp.full((L,), jnp.int32(2 ** 30), jnp.int32)
    n_groups = n // (GROUP * L)

    def start(row, buf, sem):
        pltpu.make_async_copy(d2_hbm.at[base + row], buf, sem).start()

    def wait(row, buf, sem):
        pltpu.make_async_copy(d2_hbm.at[base + row], buf, sem).wait()

    def process_row(rl, buf):
        t_vec = tv[pl.ds(rl * K, L)]
        for j in range(CAND // L):
            cv[pl.ds(j * L, L)] = inf_vec

        def scan_group(g, cnt):
            e0 = g * (GROUP * L)
            vs = [buf[pl.ds(e0 + u * L, L)] for u in range(GROUP)]
            tier = vs
            while len(tier) > 1:
                tier = [jnp.minimum(a, b) for a, b in zip(tier[::2], tier[1::2])]
            gm = tier[0]

            def do_hits(c):
                vs, ms, pcs = [], [], []
                for u in range(GROUP):
                    v = buf[pl.ds(e0 + u * L, L)]
                    m = v <= t_vec
                    vs.append(v)
                    ms.append(m)
                    pcs.append(jnp.sum(m.astype(jnp.int32), axis=0))
                for u in range(GROUP):
                    idxv = iota + (e0 + u * L)
                    off = jnp.minimum(c, CAND - L)
                    plsc.store_compressed(cv.at[pl.ds(off, L)], vs[u],
                                          mask=ms[u])
                    plsc.store_compressed(ci.at[pl.ds(off, L)], idxv,
                                          mask=ms[u])
                    c = c + pcs[u]
                return c

            return lax.cond(jnp.any(gm <= t_vec), do_hits, lambda c: c, cnt)

        cnt = lax.fori_loop(0, n_groups, scan_group, jnp.int32(0))

        def merge(j, carry):
            acc_v, acc_i = carry
            v = cv[pl.ds(j * L, L)]
            i = ci[pl.ds(j * L, L)]
            sv, si = plsc.sort_key_val(v, i)
            rv = lax.rev(sv, (0,))
            ri = lax.rev(si, (0,))
            keep = (acc_v < rv) | ((acc_v == rv) & (acc_i <= ri))
            mv = jnp.where(keep, acc_v, rv)
            mi = jnp.where(keep, acc_i, ri)
            return tuple(plsc.sort_key_val(mv, mi))

        nv = (cnt + (L - 1)) // L
        acc_v, acc_i = lax.fori_loop(0, nv, merge, (inf_vec, big_i))
        nxt = jnp.minimum(iota + 1, L - 1)
        prv = jnp.maximum(iota - 1, 0)
        vn = _gather16(acc_v, nxt)
        ix = _gather16(acc_i, nxt)
        vp = _gather16(acc_v, prv)
        ip = _gather16(acc_i, prv)
        take_next = (vn == acc_v) & (ix < acc_i)
        take_prev = (vp == acc_v) & (ip > acc_i)
        acc_i = jnp.where(take_next, ix, jnp.where(take_prev, ip, acc_i))
        ov[pl.ds(rl * K, K)] = acc_v
        oi[pl.ds(rl * K, K)] = acc_i

    start(0, buf0, sem0)
    start(1, buf1, sem1)

    def pair(g, _):
        row = 2 * g
        wait(row, buf0, sem0)
        process_row(row, buf0)

        @pl.when(g < rows_per // 2 - 1)
        def _():
            start(row + 2, buf0, sem0)

        wait(row + 1, buf1, sem1)
        process_row(row + 1, buf1)

        @pl.when(g < rows_per // 2 - 1)
        def _():
            start(row + 3, buf1, sem1)

        return 0

    lax.fori_loop(0, rows_per // 2, pair, 0)

    pltpu.sync_copy(ov, vals_hbm.at[pl.ds(base * K, rows_per * K)])
    pltpu.sync_copy(oi, idx_hbm.at[pl.ds(base * K, rows_per * K)])


def _stage_b(d2, t):
    rows, n = d2.shape
    rows_per = rows // NW
    mesh = plsc.VectorSubcoreMesh(core_axis_name="c", subcore_axis_name="s",
                                  num_cores=NC, num_subcores=NS)
    f = pl.kernel(
        _sc_body,
        out_type=[
            jax.ShapeDtypeStruct((rows * K,), jnp.float32),
            jax.ShapeDtypeStruct((rows * K,), jnp.int32),
        ],
        mesh=mesh,
        scratch_types=[
            pltpu.VMEM((rows_per * K,), jnp.float32),
            pltpu.VMEM((n,), jnp.float32),
            pltpu.VMEM((n,), jnp.float32),
            pltpu.VMEM((CAND,), jnp.float32),
            pltpu.VMEM((CAND,), jnp.int32),
            pltpu.VMEM((rows_per * K,), jnp.float32),
            pltpu.VMEM((rows_per * K,), jnp.int32),
            pltpu.SemaphoreType.DMA,
            pltpu.SemaphoreType.DMA,
        ],
        compiler_params=pltpu.CompilerParams(needs_layout_passes=False),
    )
    return f(d2, t)


def _sqrt_body(v_ref, o_ref):
    o_ref[...] = jnp.sqrt(v_ref[...])


def _stage_c(v):
    return pl.pallas_call(
        _sqrt_body,
        out_shape=jax.ShapeDtypeStruct(v.shape, jnp.float32),
    )(v)


def kernel(ref, query):
    b, n, d = ref.shape
    _, q, _ = query.shape
    h = 1
    r2 = _stage_a0(ref)
    outs = []
    for s in [slice(i, i + 1) for i in range(b)]:
        d2, t = _stage_a(ref[s], query[s], r2[s])
        outs.append(_stage_b(d2.reshape(h * q, n), t.reshape(h * q * K)))
    vals = jnp.concatenate([o[0] for o in outs])
    idxs = jnp.concatenate([o[1] for o in outs])
    dist = _stage_c(vals.reshape(b * q, K))
    return dist.reshape(b, q, K), idxs.reshape(b, q, K)


# speedup vs baseline: 1.8310x; 1.0016x over previous
import jax
import jax.numpy as jnp
from jax import lax
from jax.experimental import pallas as pl
from jax.experimental.pallas import tpu as pltpu
from jax.experimental.pallas import tpu_sc as plsc

K = 16
QB = 128
NC, NS, L = 2, 16, 16
NW = NC * NS
CAND = 128
GROUP = 16


def _sum16(x):
    v = x
    while v.shape[1] > 1:
        h = v.shape[1] // 2
        v = v[:, :h] + v[:, h:]
    return v[:, 0]


def _r2_block(ref_ref, r2_ref):
    r = ref_ref[0]
    r2_ref[0, 0] = _sum16(r * r)


def _stage_a0(ref):
    b, n, d = ref.shape
    return pl.pallas_call(
        _r2_block,
        grid=(b,),
        in_specs=[pl.BlockSpec((1, n, d), lambda bi: (bi, 0, 0))],
        out_specs=pl.BlockSpec((1, 1, n), lambda bi: (bi, 0, 0)),
        out_shape=jax.ShapeDtypeStruct((b, 1, n), jnp.float32),
    )(ref)


def _dist_block(query_ref, ref_ref, r2_ref, d2_ref, t_ref):
    q = query_ref[0]
    r = ref_ref[0]
    n = r.shape[0]
    dot = lax.dot_general(q, r, (((1,), (1,)), ((), ())),
                          preferred_element_type=jnp.float32)
    q2 = _sum16(q * q)
    r2 = r2_ref[0, 0]
    d2 = jnp.maximum((q2[:, None] + r2[None, :]) - 2.0 * dot, 0.0)
    d2_ref[0] = d2
    lm = d2[:, 0:128]
    for i in range(1, n // 128):
        lm = jnp.minimum(lm, d2[:, i * 128:(i + 1) * 128])
    inf = jnp.float32(jnp.inf)
    for _ in range(K - 1):
        m = jnp.min(lm, axis=1)
        lm = jnp.where(lm == m[:, None], inf, lm)
    t_ref[0] = jnp.broadcast_to(jnp.min(lm, axis=1)[:, None], (lm.shape[0], K))


QBA = 128


def _stage_a(ref, query, r2):
    b, n, d = ref.shape
    _, q, _ = query.shape
    return pl.pallas_call(
        _dist_block,
        grid=(b, q // QBA),
        in_specs=[
            pl.BlockSpec((1, QBA, d), lambda bi, qi: (bi, qi, 0)),
            pl.BlockSpec((1, n, d), lambda bi, qi: (bi, 0, 0)),
            pl.BlockSpec((1, 1, n), lambda bi, qi: (bi, 0, 0)),
        ],
        out_specs=[
            pl.BlockSpec((1, QBA, n), lambda bi, qi: (bi, qi, 0)),
            pl.BlockSpec((1, QBA, K), lambda bi, qi: (bi, qi, 0)),
        ],
        out_shape=[
            jax.ShapeDtypeStruct((b, q, n), jnp.float32),
            jax.ShapeDtypeStruct((b, q, K), jnp.float32),
        ],
    )(query, ref, r2)


def _gather16(x, idx):
    return lax.gather(
        x, idx[:, None],
        dimension_numbers=lax.GatherDimensionNumbers(
            offset_dims=(), collapsed_slice_dims=(0,), start_index_map=(0,)),
        slice_sizes=(1,),
        mode=lax.GatherScatterMode.PROMISE_IN_BOUNDS)


def _sc_body(d2_hbm, t_hbm, vals_hbm, idx_hbm,
             tv, buf0, buf1, cv, ci, ov, oi, sem0, sem1):
    n = d2_hbm.shape[1]
    rows_per = d2_hbm.shape[0] // NW
    wid = lax.axis_index("s") * NC + lax.axis_index("c")
    base = wid * rows_per

    pltpu.sync_copy(t_hbm.at[pl.ds(base * K, rows_per * K)], tv)

    iota = lax.broadcasted_iota(jnp.int32, (L,), 0)
    inf = jnp.float32(jnp.inf)
    inf_vec = jnp.full((L,), inf, jnp.float32)
    big_i = jnp.full((L,), jnp.int32(2 ** 30), jnp.int32)
    n_groups = n // (GROUP * L)

    def start(row, buf, sem):
        pltpu.make_async_copy(d2_hbm.at[base + row], buf, sem).start()

    def wait(row, buf, sem):
        pltpu.make_async_copy(d2_hbm.at[base + row], buf, sem).wait()

    def process_row(rl, buf):
        t_vec = tv[pl.ds(rl * K, L)]
        for j in range(CAND // L):
            cv[pl.ds(j * L, L)] = inf_vec

        def scan_group(g, cnt):
            e0 = g * (GROUP * L)
            vs = [buf[pl.ds(e0 + u * L, L)] for u in range(GROUP)]
            tier = vs
            while len(tier) > 1:
                tier = [jnp.minimum(a, b) for a, b in zip(tier[::2], tier[1::2])]
            gm = tier[0]

            def do_hits(c):
                vs, ms, pcs = [], [], []
                for u in range(GROUP):
                    v = buf[pl.ds(e0 + u * L, L)]
                    m = v <= t_vec
                    vs.append(v)
                    ms.append(m)
                    pcs.append(jnp.sum(m.astype(jnp.int32), axis=0))
                for u in range(GROUP):
                    idxv = iota + (e0 + u * L)
                    off = jnp.minimum(c, CAND - L)
                    plsc.store_compressed(cv.at[pl.ds(off, L)], vs[u],
                                          mask=ms[u])
                    plsc.store_compressed(ci.at[pl.ds(off, L)], idxv,
                                          mask=ms[u])
                    c = c + pcs[u]
                return c

            return lax.cond(jnp.any(gm <= t_vec), do_hits, lambda c: c, cnt)

        cnt = lax.fori_loop(0, n_groups, scan_group, jnp.int32(0))

        def merge(j, carry):
            acc_v, acc_i = carry
            v = cv[pl.ds(j * L, L)]
            i = ci[pl.ds(j * L, L)]
            sv, si = plsc.sort_key_val(v, i)
            rv = lax.rev(sv, (0,))
            ri = lax.rev(si, (0,))
            keep = (acc_v < rv) | ((acc_v == rv) & (acc_i <= ri))
            mv = jnp.where(keep, acc_v, rv)
            mi = jnp.where(keep, acc_i, ri)
            return tuple(plsc.sort_key_val(mv, mi))

        nv = (cnt + (L - 1)) // L
        acc_v, acc_i = lax.fori_loop(0, nv, merge, (inf_vec, big_i))
        nxt = jnp.minimum(iota + 1, L - 1)
        prv = jnp.maximum(iota - 1, 0)
        vn = _gather16(acc_v, nxt)
        ix = _gather16(acc_i, nxt)
        vp = _gather16(acc_v, prv)
        ip = _gather16(acc_i, prv)
        take_next = (vn == acc_v) & (ix < acc_i)
        take_prev = (vp == acc_v) & (ip > acc_i)
        acc_i = jnp.where(take_next, ix, jnp.where(take_prev, ip, acc_i))
        ov[pl.ds(rl * K, K)] = acc_v
        oi[pl.ds(rl * K, K)] = acc_i

    start(0, buf0, sem0)
    start(1, buf1, sem1)

    def pair(g, _):
        row = 2 * g
        wait(row, buf0, sem0)
        process_row(row, buf0)

        @pl.when(g < rows_per // 2 - 1)
        def _():
            start(row + 2, buf0, sem0)

        wait(row + 1, buf1, sem1)
        process_row(row + 1, buf1)

        @pl.when(g < rows_per // 2 - 1)
        def _():
            start(row + 3, buf1, sem1)

        return 0

    lax.fori_loop(0, rows_per // 2, pair, 0)

    pltpu.sync_copy(ov, vals_hbm.at[pl.ds(base * K, rows_per * K)])
    pltpu.sync_copy(oi, idx_hbm.at[pl.ds(base * K, rows_per * K)])


def _stage_b(d2, t):
    rows, n = d2.shape
    rows_per = rows // NW
    mesh = plsc.VectorSubcoreMesh(core_axis_name="c", subcore_axis_name="s",
                                  num_cores=NC, num_subcores=NS)
    f = pl.kernel(
        _sc_body,
        out_type=[
            jax.ShapeDtypeStruct((rows * K,), jnp.float32),
            jax.ShapeDtypeStruct((rows * K,), jnp.int32),
        ],
        mesh=mesh,
        scratch_types=[
            pltpu.VMEM((rows_per * K,), jnp.float32),
            pltpu.VMEM((n,), jnp.float32),
            pltpu.VMEM((n,), jnp.float32),
            pltpu.VMEM((CAND,), jnp.float32),
            pltpu.VMEM((CAND,), jnp.int32),
            pltpu.VMEM((rows_per * K,), jnp.float32),
            pltpu.VMEM((rows_per * K,), jnp.int32),
            pltpu.SemaphoreType.DMA,
            pltpu.SemaphoreType.DMA,
        ],
        compiler_params=pltpu.CompilerParams(needs_layout_passes=False),
    )
    return f(d2, t)


def _sqrt_body(v_ref, o_ref):
    o_ref[...] = jnp.sqrt(v_ref[...])


def _stage_c(v):
    return pl.pallas_call(
        _sqrt_body,
        out_shape=jax.ShapeDtypeStruct(v.shape, jnp.float32),
    )(v)


def kernel(ref, query):
    b, n, d = ref.shape
    _, q, _ = query.shape
    h = 1
    r2 = _stage_a0(ref)
    outs = []
    for s in [slice(i, i + 1) for i in range(b)]:
        d2, t = _stage_a(ref[s], query[s], r2[s])
        outs.append(_stage_b(d2.reshape(h * q, n), t.reshape(h * q * K)))
    vals = jnp.concatenate([o[0] for o in outs])
    idxs = jnp.concatenate([o[1] for o in outs])
    dist = _stage_c(vals.reshape(b * q, K))
    return dist.reshape(b, q, K), idxs.reshape(b, q, K)
